# baseline (speedup 1.0000x reference)
import functools

import jax
import jax.numpy as jnp
from jax import lax
from jax.experimental import pallas as pl
from jax.experimental.pallas import tpu as pltpu
from jax.experimental.pallas import tpu_sc as plsc

N = 10000
E = 320000
F = 128
EMB = 256
VOCAB = 10000

NP = 10240
CH = 128
NCH = 80
EP = 32 * NCH * CH
NTILE = 16
RT = NP // NTILE

UNITS = EP // (NTILE * CH)
U0 = 57
E0 = U0 * NTILE * CH
E1 = EP - E0
TE0 = E0 // NTILE
TE1 = E1 // NTILE
NCH0 = TE0 // CH
NCH1 = TE1 // CH


def _sc_mesh():
    return plsc.VectorSubcoreMesh(core_axis_name="c", subcore_axis_name="s")


def _deg_partials(dstp, zinit, ones):
    @functools.partial(
        pl.kernel,
        out_type=jax.ShapeDtypeStruct((2, NP, F), jnp.float32),
        mesh=_sc_mesh(),
        scratch_types=[
            pltpu.VMEM((NCH, CH), jnp.int32),
            pltpu.VMEM((CH, F), jnp.float32),
            pltpu.VMEM_SHARED((NP, F), jnp.float32),
        ],
    )
    def k(dst_hbm, z_hbm, ones_hbm, out_hbm, dall, ones_v, acc_sh):
        cid = lax.axis_index("c")
        sid = lax.axis_index("s")
        wid = cid * 16 + sid
        r0 = sid * RT
        pltpu.sync_copy(z_hbm.at[pl.ds(r0, RT)], acc_sh.at[pl.ds(r0, RT)])
        pltpu.sync_copy(ones_hbm, ones_v)
        pltpu.sync_copy(dst_hbm.at[wid], dall)
        plsc.subcore_barrier()

        def body(i, carry):
            pltpu.sync_copy(ones_v, acc_sh.at[dall.at[i]], add=True)
            return carry

        lax.fori_loop(0, NCH, body, 0)
        plsc.subcore_barrier()
        pltpu.sync_copy(acc_sh.at[pl.ds(r0, RT)], out_hbm.at[cid, pl.ds(r0, RT)])

    return k(dstp, zinit, ones)


def _aggregate(y, srcp, dstp, init):
    @functools.partial(
        pl.kernel,
        out_type=jax.ShapeDtypeStruct((2, NP, F), jnp.float32),
        mesh=_sc_mesh(),
        scratch_types=[
            pltpu.VMEM((CH,), jnp.int32),
            pltpu.VMEM((CH,), jnp.int32),
            pltpu.VMEM((CH, F), jnp.float32),
            pltpu.VMEM_SHARED((NP, F), jnp.float32),
            pltpu.SemaphoreType.DMA,
        ],
    )
    def k(y_hbm, src_hbm, dst_hbm, init_hbm, out_hbm, sidx, didx, rows, acc_sh, sem):
        cid = lax.axis_index("c")
        sid = lax.axis_index("s")
        r0 = sid * RT
        pltpu.sync_copy(init_hbm.at[cid, pl.ds(r0, RT)], acc_sh.at[pl.ds(r0, RT)])
        plsc.subcore_barrier()
        te = jnp.where(cid == 0, TE0, TE1)
        nch = jnp.where(cid == 0, NCH0, NCH1)
        base = cid * E0 + sid * te

        def body(i, carry):
            off = base + i * CH
            pltpu.sync_copy(src_hbm.at[pl.ds(off, CH)], sidx)
            pltpu.sync_copy(dst_hbm.at[pl.ds(off, CH)], didx)
            pltpu.async_copy(y_hbm.at[sidx], rows, sem).wait()
            pltpu.sync_copy(rows, acc_sh.at[didx], add=True)
            return carry

        lax.fori_loop(0, nch, body, 0)
        plsc.subcore_barrier()
        pltpu.sync_copy(acc_sh.at[pl.ds(r0, RT)], out_hbm.at[cid, pl.ds(r0, RT)])

    return k(y, srcp, dstp, init)


def _cdiv(a, b):
    return (a + b - 1) // b


def _mm_scale(x, W1, deg_p):
    bm = 512
    grid = (_cdiv(NP, bm),)

    def body(x_ref, w_ref, d_ref, y_ref):
        dinv = lax.rsqrt(d_ref[0, :, 0:1] + d_ref[1, :, 0:1] + 1.0)
        xw = jnp.dot(x_ref[...], w_ref[...], preferred_element_type=jnp.float32)
        y_ref[...] = xw * dinv

    return pl.pallas_call(
        body,
        grid=grid,
        in_specs=[
            pl.BlockSpec((bm, F), lambda i: (i, 0)),
            pl.BlockSpec((F, F), lambda i: (0, 0)),
            pl.BlockSpec((2, bm, F), lambda i: (0, i, 0)),
        ],
        out_specs=pl.BlockSpec((bm, F), lambda i: (i, 0)),
        out_shape=jax.ShapeDtypeStruct((NP, F), jnp.float32),
    )(x, W1, deg_p)


def _post_mm(acc1, deg_p, b1r, W2):
    bm = 512
    grid = (_cdiv(NP, bm),)

    def body(a_ref, d_ref, b_ref, w_ref, y_ref):
        dinv = lax.rsqrt(d_ref[0, :, 0:1] + d_ref[1, :, 0:1] + 1.0)
        accf = a_ref[0] + a_ref[1]
        h = jnp.maximum(accf * dinv + b_ref[...], 0.0)
        y_ref[...] = jnp.dot(h, w_ref[...], preferred_element_type=jnp.float32) * dinv

    return pl.pallas_call(
        body,
        grid=grid,
        in_specs=[
            pl.BlockSpec((2, bm, F), lambda i: (0, i, 0)),
            pl.BlockSpec((2, bm, F), lambda i: (0, i, 0)),
            pl.BlockSpec((1, F), lambda i: (0, 0)),
            pl.BlockSpec((F, F), lambda i: (0, 0)),
        ],
        out_specs=pl.BlockSpec((bm, F), lambda i: (i, 0)),
        out_shape=jax.ShapeDtypeStruct((NP, F), jnp.float32),
    )(acc1, deg_p, b1r, W2)


def _final(acc2, deg_p, b2r, W_ih, bgr, W_fc, bfcr):
    bm = 256
    bn = 1024
    grid = (_cdiv(N, bm), _cdiv(VOCAB, bn))

    def body(a_ref, d_ref, b2_ref, wih_ref, bg_ref, wfc_ref, bfc_ref,
             out_ref, hh_ref):
        j = pl.program_id(1)

        @pl.when(j == 0)
        def _():
            dinv = lax.rsqrt(d_ref[0, :, 0:1] + d_ref[1, :, 0:1] + 1.0)
            h2 = (a_ref[0] + a_ref[1]) * dinv + b2_ref[...]
            gates = lax.dot_general(
                h2, wih_ref[...], (((1,), (1,)), ((), ())),
                preferred_element_type=jnp.float32) + bg_ref[...]
            ii = jax.nn.sigmoid(gates[:, 0:EMB])
            gg = jnp.tanh(gates[:, 2 * EMB:3 * EMB])
            oo = jax.nn.sigmoid(gates[:, 3 * EMB:4 * EMB])
            hh_ref[...] = oo * jnp.tanh(ii * gg)

        out_ref[...] = lax.dot_general(
            hh_ref[...], wfc_ref[...], (((1,), (1,)), ((), ())),
            preferred_element_type=jnp.float32) + bfc_ref[...]

    return pl.pallas_call(
        body,
        grid=grid,
        in_specs=[
            pl.BlockSpec((2, bm, F), lambda i, j: (0, i, 0)),
            pl.BlockSpec((2, bm, F), lambda i, j: (0, i, 0)),
            pl.BlockSpec((1, F), lambda i, j: (0, 0)),
            pl.BlockSpec((4 * EMB, F), lambda i, j: (0, 0)),
            pl.BlockSpec((1, 4 * EMB), lambda i, j: (0, 0)),
            pl.BlockSpec((bn, EMB), lambda i, j: (j, 0)),
            pl.BlockSpec((1, bn), lambda i, j: (0, j)),
        ],
        out_specs=pl.BlockSpec((bm, bn), lambda i, j: (i, j)),
        out_shape=jax.ShapeDtypeStruct((N, VOCAB), jnp.float32),
        scratch_shapes=[pltpu.VMEM((bm, EMB), jnp.float32)],
    )(acc2, deg_p, b2r, W_ih, bgr, W_fc, bfcr)


def kernel(x, edge_index, W1, b1, W2, b2, W_ih, W_hh, b_ih, b_hh, W_fc, b_fc):
    src = edge_index[0].astype(jnp.int32)
    dst = edge_index[1].astype(jnp.int32)
    pad = EP - E
    srcp = jnp.concatenate([src, jnp.full((pad,), N, jnp.int32)])
    dstp = jnp.concatenate(
        [dst, N + (jnp.arange(pad, dtype=jnp.int32) % 16)])

    zinit = jnp.zeros((NP, F), jnp.float32)
    ones = jnp.ones((CH, F), jnp.float32)
    deg_p = _deg_partials(dstp.reshape(32, NCH, CH), zinit, ones)

    zacc = jnp.zeros((1, NP, F), jnp.float32)
    y1 = _mm_scale(x, W1, deg_p)
    acc1 = _aggregate(y1, srcp, dstp,
                      jnp.concatenate([y1[None], zacc], axis=0))

    y2 = _post_mm(acc1, deg_p, b1.reshape(1, F), W2)
    acc2 = _aggregate(y2, srcp, dstp,
                      jnp.concatenate([y2[None], zacc], axis=0))

    logits = _final(acc2, deg_p, b2.reshape(1, F), W_ih,
                    (b_ih + b_hh).reshape(1, 4 * EMB), W_fc,
                    b_fc.reshape(1, VOCAB))
    return logits


# speedup vs baseline: 1.5475x; 1.5475x over previous
import functools

import jax
import jax.numpy as jnp
from jax import lax
from jax.experimental import pallas as pl
from jax.experimental.pallas import tpu as pltpu
from jax.experimental.pallas import tpu_sc as plsc

N = 10000
E = 320000
F = 128
EMB = 256
VOCAB = 10000

NP = 10240
CH = 128
NCH = 79
EP = 32 * NCH * CH
NTILE = 16
RT = NP // NTILE
TE = EP // 32


def _sc_mesh():
    return plsc.VectorSubcoreMesh(core_axis_name="c", subcore_axis_name="s")


def _deg_partials(dstp, zinit, ones):
    @functools.partial(
        pl.kernel,
        out_type=jax.ShapeDtypeStruct((2, NP, F), jnp.float32),
        mesh=_sc_mesh(),
        scratch_types=[
            pltpu.VMEM((NCH, CH), jnp.int32),
            pltpu.VMEM((CH, F), jnp.float32),
            pltpu.VMEM_SHARED((NP, F), jnp.float32),
        ],
    )
    def k(dst_hbm, z_hbm, ones_hbm, out_hbm, dall, ones_v, acc_sh):
        cid = lax.axis_index("c")
        sid = lax.axis_index("s")
        wid = cid * 16 + sid
        r0 = sid * RT
        pltpu.sync_copy(z_hbm.at[pl.ds(r0, RT)], acc_sh.at[pl.ds(r0, RT)])
        pltpu.sync_copy(ones_hbm, ones_v)
        pltpu.sync_copy(dst_hbm.at[wid], dall)
        plsc.subcore_barrier()

        def body(i, carry):
            pltpu.sync_copy(ones_v, acc_sh.at[dall.at[i]], add=True)
            return carry

        lax.fori_loop(0, NCH, body, 0)
        plsc.subcore_barrier()
        pltpu.sync_copy(acc_sh.at[pl.ds(r0, RT)], out_hbm.at[cid, pl.ds(r0, RT)])

    return k(dstp, zinit, ones)


def _aggregate(y, srcp, dstp, init):
    @functools.partial(
        pl.kernel,
        out_type=jax.ShapeDtypeStruct((2, NP, F), jnp.float32),
        mesh=_sc_mesh(),
        scratch_types=[
            pltpu.VMEM((CH,), jnp.int32),
            pltpu.VMEM((CH,), jnp.int32),
            pltpu.VMEM((CH, F), jnp.float32),
            pltpu.VMEM_SHARED((NP, F), jnp.float32),
            pltpu.SemaphoreType.DMA,
        ],
    )
    def k(y_hbm, src_hbm, dst_hbm, init_hbm, out_hbm, sidx, didx, rows, acc_sh, sem):
        cid = lax.axis_index("c")
        sid = lax.axis_index("s")
        r0 = sid * RT
        pltpu.sync_copy(init_hbm.at[cid, pl.ds(r0, RT)], acc_sh.at[pl.ds(r0, RT)])
        plsc.subcore_barrier()
        base = (cid * 16 + sid) * TE

        def body(i, carry):
            off = base + i * CH
            pltpu.sync_copy(src_hbm.at[pl.ds(off, CH)], sidx)
            pltpu.sync_copy(dst_hbm.at[pl.ds(off, CH)], didx)
            pltpu.async_copy(y_hbm.at[sidx], rows, sem).wait()
            pltpu.sync_copy(rows, acc_sh.at[didx], add=True)
            return carry

        lax.fori_loop(0, NCH, body, 0)
        plsc.subcore_barrier()
        pltpu.sync_copy(acc_sh.at[pl.ds(r0, RT)], out_hbm.at[cid, pl.ds(r0, RT)])

    return k(y, srcp, dstp, init)


def _cdiv(a, b):
    return (a + b - 1) // b


def _mm_scale(x, W1, deg_p):
    bm = 512
    grid = (_cdiv(NP, bm),)

    def body(x_ref, w_ref, d_ref, y_ref):
        dinv = lax.rsqrt(d_ref[0, :, 0:1] + d_ref[1, :, 0:1] + 1.0)
        xw = jnp.dot(x_ref[...], w_ref[...], preferred_element_type=jnp.float32)
        y_ref[...] = xw * dinv

    return pl.pallas_call(
        body,
        grid=grid,
        in_specs=[
            pl.BlockSpec((bm, F), lambda i: (i, 0)),
            pl.BlockSpec((F, F), lambda i: (0, 0)),
            pl.BlockSpec((2, bm, F), lambda i: (0, i, 0)),
        ],
        out_specs=pl.BlockSpec((bm, F), lambda i: (i, 0)),
        out_shape=jax.ShapeDtypeStruct((NP, F), jnp.float32),
    )(x, W1, deg_p)


def _post_mm(acc1, deg_p, b1r, W2):
    bm = 512
    grid = (_cdiv(NP, bm),)

    def body(a_ref, d_ref, b_ref, w_ref, y_ref):
        dinv = lax.rsqrt(d_ref[0, :, 0:1] + d_ref[1, :, 0:1] + 1.0)
        accf = a_ref[0] + a_ref[1]
        h = jnp.maximum(accf * dinv + b_ref[...], 0.0)
        y_ref[...] = jnp.dot(h, w_ref[...], preferred_element_type=jnp.float32) * dinv

    return pl.pallas_call(
        body,
        grid=grid,
        in_specs=[
            pl.BlockSpec((2, bm, F), lambda i: (0, i, 0)),
            pl.BlockSpec((2, bm, F), lambda i: (0, i, 0)),
            pl.BlockSpec((1, F), lambda i: (0, 0)),
            pl.BlockSpec((F, F), lambda i: (0, 0)),
        ],
        out_specs=pl.BlockSpec((bm, F), lambda i: (i, 0)),
        out_shape=jax.ShapeDtypeStruct((NP, F), jnp.float32),
    )(acc1, deg_p, b1r, W2)


def _lstm(acc2, deg_p, b2r, W_ih, bgr):
    bm = 512
    grid = (_cdiv(NP, bm),)

    def body(a_ref, d_ref, b2_ref, wih_ref, bg_ref, out_ref):
        dinv = lax.rsqrt(d_ref[0, :, 0:1] + d_ref[1, :, 0:1] + 1.0)
        h2 = (a_ref[0] + a_ref[1]) * dinv + b2_ref[...]
        gates = lax.dot_general(
            h2, wih_ref[...], (((1,), (1,)), ((), ())),
            preferred_element_type=jnp.float32) + bg_ref[...]
        ii = jax.nn.sigmoid(gates[:, 0:EMB])
        gg = jnp.tanh(gates[:, 2 * EMB:3 * EMB])
        oo = jax.nn.sigmoid(gates[:, 3 * EMB:4 * EMB])
        out_ref[...] = (oo * jnp.tanh(ii * gg)).astype(jnp.bfloat16)

    return pl.pallas_call(
        body,
        grid=grid,
        in_specs=[
            pl.BlockSpec((2, bm, F), lambda i: (0, i, 0)),
            pl.BlockSpec((2, bm, F), lambda i: (0, i, 0)),
            pl.BlockSpec((1, F), lambda i: (0, 0)),
            pl.BlockSpec((4 * EMB, F), lambda i: (0, 0)),
            pl.BlockSpec((1, 4 * EMB), lambda i: (0, 0)),
        ],
        out_specs=pl.BlockSpec((bm, EMB), lambda i: (i, 0)),
        out_shape=jax.ShapeDtypeStruct((NP, EMB), jnp.bfloat16),
    )(acc2, deg_p, b2r, W_ih, bgr)


def _vocab(hh, W_fc_bf, bfcr):
    bm = 512
    bn = 1024
    grid = (_cdiv(VOCAB, bn), _cdiv(N, bm))

    def body(h_ref, w_ref, b_ref, out_ref):
        out_ref[...] = lax.dot_general(
            h_ref[...], w_ref[...], (((1,), (1,)), ((), ())),
            preferred_element_type=jnp.float32) + b_ref[...]

    return pl.pallas_call(
        body,
        grid=grid,
        in_specs=[
            pl.BlockSpec((bm, EMB), lambda j, i: (i, 0)),
            pl.BlockSpec((bn, EMB), lambda j, i: (j, 0)),
            pl.BlockSpec((1, bn), lambda j, i: (0, j)),
        ],
        out_specs=pl.BlockSpec((bm, bn), lambda j, i: (i, j)),
        out_shape=jax.ShapeDtypeStruct((N, VOCAB), jnp.float32),
    )(hh, W_fc_bf, bfcr)


def kernel(x, edge_index, W1, b1, W2, b2, W_ih, W_hh, b_ih, b_hh, W_fc, b_fc):
    src = edge_index[0].astype(jnp.int32)
    dst = edge_index[1].astype(jnp.int32)
    pad = EP - E
    srcp = jnp.concatenate([src, jnp.full((pad,), N, jnp.int32)])
    dstp = jnp.concatenate(
        [dst, N + (jnp.arange(pad, dtype=jnp.int32) % 16)])

    zinit = jnp.zeros((NP, F), jnp.float32)
    ones = jnp.ones((CH, F), jnp.float32)
    deg_p = _deg_partials(dstp.reshape(32, NCH, CH), zinit, ones)

    zacc = jnp.zeros((1, NP, F), jnp.float32)
    y1 = _mm_scale(x, W1, deg_p)
    acc1 = _aggregate(y1, srcp, dstp,
                      jnp.concatenate([y1[None], zacc], axis=0))

    y2 = _post_mm(acc1, deg_p, b1.reshape(1, F), W2)
    acc2 = _aggregate(y2, srcp, dstp,
                      jnp.concatenate([y2[None], zacc], axis=0))

    hh = _lstm(acc2, deg_p, b2.reshape(1, F), W_ih,
               (b_ih + b_hh).reshape(1, 4 * EMB))
    logits = _vocab(hh, W_fc.astype(jnp.bfloat16), b_fc.reshape(1, VOCAB))
    return logits


# speedup vs baseline: 1.8532x; 1.1975x over previous
import functools

import jax
import jax.numpy as jnp
from jax import lax
from jax.experimental import pallas as pl
from jax.experimental.pallas import tpu as pltpu
from jax.experimental.pallas import tpu_sc as plsc

N = 10000
E = 320000
F = 128
EMB = 256
VOCAB = 10000

NP = 10240
CH = 128
NCH = 79
EP = 32 * NCH * CH
NTILE = 16
RT = NP // NTILE
TE = EP // 32


def _sc_mesh():
    return plsc.VectorSubcoreMesh(core_axis_name="c", subcore_axis_name="s")


def _deg_partials(dstp, zinit, ones):
    @functools.partial(
        pl.kernel,
        out_type=jax.ShapeDtypeStruct((2, NP, F), jnp.float32),
        mesh=_sc_mesh(),
        scratch_types=[
            pltpu.VMEM((NCH, CH), jnp.int32),
            pltpu.VMEM((CH, F), jnp.float32),
            pltpu.VMEM_SHARED((NP, F), jnp.float32),
        ],
    )
    def k(dst_hbm, z_hbm, ones_hbm, out_hbm, dall, ones_v, acc_sh):
        cid = lax.axis_index("c")
        sid = lax.axis_index("s")
        wid = cid * 16 + sid
        r0 = sid * RT
        pltpu.sync_copy(z_hbm.at[pl.ds(r0, RT)], acc_sh.at[pl.ds(r0, RT)])
        pltpu.sync_copy(ones_hbm, ones_v)
        pltpu.sync_copy(dst_hbm.at[wid], dall)
        plsc.subcore_barrier()

        def body(i, carry):
            pltpu.sync_copy(ones_v, acc_sh.at[dall.at[i]], add=True)
            return carry

        lax.fori_loop(0, NCH, body, 0)
        plsc.subcore_barrier()
        pltpu.sync_copy(acc_sh.at[pl.ds(r0, RT)], out_hbm.at[cid, pl.ds(r0, RT)])

    return k(dstp, zinit, ones)


def _aggregate(y, srcp, dstp, init):
    @functools.partial(
        pl.kernel,
        out_type=jax.ShapeDtypeStruct((2, NP, F), jnp.float32),
        mesh=_sc_mesh(),
        scratch_types=[
            pltpu.VMEM((CH,), jnp.int32),
            pltpu.VMEM((CH,), jnp.int32),
            pltpu.VMEM((CH,), jnp.int32),
            pltpu.VMEM((CH,), jnp.int32),
            pltpu.VMEM((CH, F), jnp.float32),
            pltpu.VMEM((CH, F), jnp.float32),
            pltpu.VMEM_SHARED((NP, F), jnp.float32),
            pltpu.SemaphoreType.DMA,
            pltpu.SemaphoreType.DMA,
        ],
    )
    def k(y_hbm, src_hbm, dst_hbm, init_hbm, out_hbm,
          sidx0, sidx1, didx0, didx1, rows0, rows1, acc_sh, sem0, sem1):
        sidx = (sidx0, sidx1)
        didx = (didx0, didx1)
        rows = (rows0, rows1)
        sems = (sem0, sem1)
        cid = lax.axis_index("c")
        sid = lax.axis_index("s")
        r0 = sid * RT
        pltpu.sync_copy(init_hbm.at[cid, pl.ds(r0, RT)], acc_sh.at[pl.ds(r0, RT)])
        plsc.subcore_barrier()
        base = (cid * 16 + sid) * TE

        for b in range(2):
            off = base + b * CH
            pltpu.sync_copy(src_hbm.at[pl.ds(off, CH)], sidx[b])
            pltpu.sync_copy(dst_hbm.at[pl.ds(off, CH)], didx[b])
            pltpu.async_copy(y_hbm.at[sidx[b]], rows[b], sems[b])

        def body(g, carry):
            for b in range(2):
                i = 2 * g + b
                pltpu.make_async_copy(y_hbm.at[sidx[b]], rows[b],
                                      sems[b]).wait()
                pltpu.sync_copy(rows[b], acc_sh.at[didx[b]], add=True)

                @pl.when(i + 2 < NCH)
                def _():
                    off = base + (i + 2) * CH
                    pltpu.sync_copy(src_hbm.at[pl.ds(off, CH)], sidx[b])
                    pltpu.sync_copy(dst_hbm.at[pl.ds(off, CH)], didx[b])
                    pltpu.async_copy(y_hbm.at[sidx[b]], rows[b], sems[b])

            return carry

        lax.fori_loop(0, NCH // 2, body, 0)
        b_last = (NCH - 1) % 2
        pltpu.make_async_copy(y_hbm.at[sidx[b_last]], rows[b_last],
                              sems[b_last]).wait()
        pltpu.sync_copy(rows[b_last], acc_sh.at[didx[b_last]], add=True)
        plsc.subcore_barrier()
        pltpu.sync_copy(acc_sh.at[pl.ds(r0, RT)], out_hbm.at[cid, pl.ds(r0, RT)])

    return k(y, srcp, dstp, init)


def _cdiv(a, b):
    return (a + b - 1) // b


def _mm_scale(x, W1, deg_p):
    bm = 512
    grid = (_cdiv(NP, bm),)

    def body(x_ref, w_ref, d_ref, y_ref):
        dinv = lax.rsqrt(d_ref[0, :, 0:1] + d_ref[1, :, 0:1] + 1.0)
        xw = jnp.dot(x_ref[...], w_ref[...], preferred_element_type=jnp.float32)
        y_ref[...] = xw * dinv

    return pl.pallas_call(
        body,
        grid=grid,
        in_specs=[
            pl.BlockSpec((bm, F), lambda i: (i, 0)),
            pl.BlockSpec((F, F), lambda i: (0, 0)),
            pl.BlockSpec((2, bm, F), lambda i: (0, i, 0)),
        ],
        out_specs=pl.BlockSpec((bm, F), lambda i: (i, 0)),
        out_shape=jax.ShapeDtypeStruct((NP, F), jnp.float32),
    )(x, W1, deg_p)


def _post_mm(acc1, deg_p, b1r, W2):
    bm = 512
    grid = (_cdiv(NP, bm),)

    def body(a_ref, d_ref, b_ref, w_ref, y_ref):
        dinv = lax.rsqrt(d_ref[0, :, 0:1] + d_ref[1, :, 0:1] + 1.0)
        accf = a_ref[0] + a_ref[1]
        h = jnp.maximum(accf * dinv + b_ref[...], 0.0)
        y_ref[...] = jnp.dot(h, w_ref[...], preferred_element_type=jnp.float32) * dinv

    return pl.pallas_call(
        body,
        grid=grid,
        in_specs=[
            pl.BlockSpec((2, bm, F), lambda i: (0, i, 0)),
            pl.BlockSpec((2, bm, F), lambda i: (0, i, 0)),
            pl.BlockSpec((1, F), lambda i: (0, 0)),
            pl.BlockSpec((F, F), lambda i: (0, 0)),
        ],
        out_specs=pl.BlockSpec((bm, F), lambda i: (i, 0)),
        out_shape=jax.ShapeDtypeStruct((NP, F), jnp.float32),
    )(acc1, deg_p, b1r, W2)


def _lstm(acc2, deg_p, b2r, W_ih, bgr):
    bm = 512
    grid = (_cdiv(NP, bm),)

    def body(a_ref, d_ref, b2_ref, wih_ref, bg_ref, out_ref):
        dinv = lax.rsqrt(d_ref[0, :, 0:1] + d_ref[1, :, 0:1] + 1.0)
        h2 = (a_ref[0] + a_ref[1]) * dinv + b2_ref[...]
        gates = lax.dot_general(
            h2, wih_ref[...], (((1,), (1,)), ((), ())),
            preferred_element_type=jnp.float32) + bg_ref[...]
        ii = jax.nn.sigmoid(gates[:, 0:EMB])
        gg = jnp.tanh(gates[:, 2 * EMB:3 * EMB])
        oo = jax.nn.sigmoid(gates[:, 3 * EMB:4 * EMB])
        out_ref[...] = (oo * jnp.tanh(ii * gg)).astype(jnp.bfloat16)

    return pl.pallas_call(
        body,
        grid=grid,
        in_specs=[
            pl.BlockSpec((2, bm, F), lambda i: (0, i, 0)),
            pl.BlockSpec((2, bm, F), lambda i: (0, i, 0)),
            pl.BlockSpec((1, F), lambda i: (0, 0)),
            pl.BlockSpec((4 * EMB, F), lambda i: (0, 0)),
            pl.BlockSpec((1, 4 * EMB), lambda i: (0, 0)),
        ],
        out_specs=pl.BlockSpec((bm, EMB), lambda i: (i, 0)),
        out_shape=jax.ShapeDtypeStruct((NP, EMB), jnp.bfloat16),
    )(acc2, deg_p, b2r, W_ih, bgr)


def _vocab(hh, W_fc_bf, bfcr):
    bm = 512
    bn = 1024
    grid = (_cdiv(VOCAB, bn), _cdiv(N, bm))

    def body(h_ref, w_ref, b_ref, out_ref):
        out_ref[...] = lax.dot_general(
            h_ref[...], w_ref[...], (((1,), (1,)), ((), ())),
            preferred_element_type=jnp.float32) + b_ref[...]

    return pl.pallas_call(
        body,
        grid=grid,
        in_specs=[
            pl.BlockSpec((bm, EMB), lambda j, i: (i, 0)),
            pl.BlockSpec((bn, EMB), lambda j, i: (j, 0)),
            pl.BlockSpec((1, bn), lambda j, i: (0, j)),
        ],
        out_specs=pl.BlockSpec((bm, bn), lambda j, i: (i, j)),
        out_shape=jax.ShapeDtypeStruct((N, VOCAB), jnp.float32),
    )(hh, W_fc_bf, bfcr)


def kernel(x, edge_index, W1, b1, W2, b2, W_ih, W_hh, b_ih, b_hh, W_fc, b_fc):
    src = edge_index[0].astype(jnp.int32)
    dst = edge_index[1].astype(jnp.int32)
    pad = EP - E
    srcp = jnp.concatenate([src, jnp.full((pad,), N, jnp.int32)])
    dstp = jnp.concatenate(
        [dst, N + (jnp.arange(pad, dtype=jnp.int32) % 16)])

    zinit = jnp.zeros((NP, F), jnp.float32)
    ones = jnp.ones((CH, F), jnp.float32)
    deg_p = _deg_partials(dstp.reshape(32, NCH, CH), zinit, ones)

    zacc = jnp.zeros((1, NP, F), jnp.float32)
    y1 = _mm_scale(x, W1, deg_p)
    acc1 = _aggregate(y1, srcp, dstp,
                      jnp.concatenate([y1[None], zacc], axis=0))

    y2 = _post_mm(acc1, deg_p, b1.reshape(1, F), W2)
    acc2 = _aggregate(y2, srcp, dstp,
                      jnp.concatenate([y2[None], zacc], axis=0))

    hh = _lstm(acc2, deg_p, b2.reshape(1, F), W_ih,
               (b_ih + b_hh).reshape(1, 4 * EMB))
    logits = _vocab(hh, W_fc.astype(jnp.bfloat16), b_fc.reshape(1, VOCAB))
    return logits
